# Initial kernel scaffold; baseline (speedup 1.0000x reference)
import jax
import jax.numpy as jnp
from jax.experimental import pallas as pl


def kernel(X, edge_index, edge_attr, params):
    raise NotImplementedError("write your pallas kernel here")


# speedup vs baseline: 2.8875x; 2.8875x over previous
import functools

import jax
import jax.numpy as jnp
from jax import lax
from jax.experimental import pallas as pl
from jax.experimental.pallas import tpu as pltpu
from jax.experimental.pallas import tpu_sc as plsc

_N = 10000
_E = 320000
_D = 128
_EPS = 1e-5

_NC = 2
_NS = 16
_NW = _NC * _NS
_EPW = _E // _NW
_K = 80
_NCH = _EPW // _K
_RPT = 624
_RTAIL = _N - _RPT * _NS


def _ln(x, g, b):
    mu = jnp.mean(x, axis=-1, keepdims=True)
    xc = x - mu
    var = jnp.mean(xc * xc, axis=-1, keepdims=True)
    return xc * lax.rsqrt(var + _EPS) * g + b


def _dot(a, w):
    return jnp.dot(a, w, preferred_element_type=jnp.float32)


def _block(h, g, e, relu=True):
    if relu:
        h = jnp.maximum(h, 0.0)
    return _ln(h, g, e)


def _tail2(h, w2, b2, g2, e2, w3, b3, g3, e3):
    h = _block(_dot(h, w2[...]) + b2[...], g2[...], e2[...])
    return _block(_dot(h, w3[...]) + b3[...], g3[...], e3[...], relu=False)


def _edge_enc_body(x_ref, w1, b1, g1, e1, w2, b2, g2, e2, w3, b3, g3, e3,
                   o_ref):
    h = _block(_dot(x_ref[...], w1[...]) + b1[...], g1[...], e1[...])
    o_ref[...] = _tail2(h, w2, b2, g2, e2, w3, b3, g3, e3)


def _node_enc_body(x_ref, w1r, b1, g1, e1, w2, b2, g2, e2, w3, b3, g3, e3,
                   wpa, wpb, xh_ref, p_ref, q_ref):
    x0 = x_ref[...][:, 0:1]
    h = _block(x0 * w1r[...] + b1[...], g1[...], e1[...])
    xh = _tail2(h, w2, b2, g2, e2, w3, b3, g3, e3)
    xh_ref[...] = xh
    p_ref[...] = _dot(xh, wpa[...])
    q_ref[...] = _dot(xh, wpb[...])


def _edge_mlp_body(gs_ref, gd_ref, eh_ref, w1c, b1, g1, e1, w2, b2, g2, e2,
                   w3, b3, g3, e3, o_ref):
    eh = eh_ref[...]
    h = _block(gs_ref[...] + gd_ref[...] + _dot(eh, w1c[...]) + b1[...],
               g1[...], e1[...])
    o_ref[...] = eh + _tail2(h, w2, b2, g2, e2, w3, b3, g3, e3)


def _node_mlp_body(xh_ref, aa_ref, ab_ref, w1a, w1b, b1, g1, e1, w2, b2, g2,
                   e2, w3, b3, g3, e3, wpa, wpb, xh_ref_o, p_ref, q_ref):
    xh = xh_ref[...]
    agg = aa_ref[...] + ab_ref[...]
    h = _block(_dot(xh, w1a[...]) + _dot(agg, w1b[...]) + b1[...],
               g1[...], e1[...])
    xh2 = xh + _tail2(h, w2, b2, g2, e2, w3, b3, g3, e3)
    xh_ref_o[...] = xh2
    p_ref[...] = _dot(xh2, wpa[...])
    q_ref[...] = _dot(xh2, wpb[...])


def _node_dec_body(xh_ref, aa_ref, ab_ref, w1a, w1b, b1, g1, e1, w2, b2, g2,
                   e2, w3, b3, g3, e3, dw1, db1, dg1, de1, dw2, db2, dg2, de2,
                   dw3, db3, y_ref):
    xh = xh_ref[...]
    agg = aa_ref[...] + ab_ref[...]
    h = _block(_dot(xh, w1a[...]) + _dot(agg, w1b[...]) + b1[...],
               g1[...], e1[...])
    xh2 = xh + _tail2(h, w2, b2, g2, e2, w3, b3, g3, e3)
    h = _block(_dot(xh2, dw1[...]) + db1[...], dg1[...], de1[...])
    h = _block(_dot(h, dw2[...]) + db2[...], dg2[...], de2[...])
    y_ref[...] = _dot(h, dw3[...]) + db3[...]


def _rowwise_call(body, nrows, rblk, data, weights, out_shapes):
    grid = (nrows // rblk,)
    in_specs = (
        [pl.BlockSpec((rblk, a.shape[1]), lambda i: (i, 0)) for a in data]
        + [pl.BlockSpec(w.shape, functools.partial(lambda nd, i: (0,) * nd,
                                                   w.ndim))
           for w in weights])
    out_specs = [pl.BlockSpec((rblk, s[1]), lambda i: (i, 0))
                 for s in out_shapes]
    out_shape = [jax.ShapeDtypeStruct(s, jnp.float32) for s in out_shapes]
    if len(out_shapes) == 1:
        out_specs, out_shape = out_specs[0], out_shape[0]
    return pl.pallas_call(
        body, grid=grid, in_specs=in_specs, out_specs=out_specs,
        out_shape=out_shape)(*data, *weights)


@functools.cache
def _sc_gather_kernel():
    mesh = plsc.VectorSubcoreMesh(core_axis_name="c", subcore_axis_name="s")

    @functools.partial(
        pl.kernel,
        mesh=mesh,
        out_type=[jax.ShapeDtypeStruct((_E, _D), jnp.float32),
                  jax.ShapeDtypeStruct((_E, _D), jnp.float32)],
        scratch_types=[pltpu.VMEM((_K,), jnp.int32),
                       pltpu.VMEM((_K, _D), jnp.float32),
                       pltpu.SemaphoreType.DMA],
    )
    def _sc_gather(src, dst, p, q, gs, gd, idx_v, rows_v, sem):
        wid = lax.axis_index("s") * _NC + lax.axis_index("c")
        base = wid * _EPW

        def chunk(c, carry):
            off = base + c * _K
            pltpu.sync_copy(src.at[pl.ds(off, _K)], idx_v)
            pltpu.async_copy(p.at[idx_v], rows_v, sem).wait()
            pltpu.sync_copy(rows_v, gs.at[pl.ds(off, _K)])
            pltpu.sync_copy(dst.at[pl.ds(off, _K)], idx_v)
            pltpu.async_copy(q.at[idx_v], rows_v, sem).wait()
            pltpu.sync_copy(rows_v, gd.at[pl.ds(off, _K)])
            return carry

        lax.fori_loop(0, _NCH, chunk, 0)

    return _sc_gather


@functools.cache
def _sc_scatter_kernel():
    mesh = plsc.VectorSubcoreMesh(core_axis_name="c", subcore_axis_name="s")

    @functools.partial(
        pl.kernel,
        mesh=mesh,
        out_type=jax.ShapeDtypeStruct((2 * _N, _D), jnp.float32),
        scratch_types=[pltpu.VMEM((_K,), jnp.int32),
                       pltpu.VMEM((_K, _D), jnp.float32),
                       pltpu.VMEM_SHARED((_N, _D), jnp.float32)],
    )
    def _sc_scatter(eh, dstids, zeros, out, idx_v, rows_v, acc):
        cid = lax.axis_index("c")
        sid = lax.axis_index("s")
        wid = sid * _NC + cid
        pltpu.sync_copy(zeros.at[pl.ds(sid * _RPT, _RPT)],
                        acc.at[pl.ds(sid * _RPT, _RPT)])

        @pl.when(sid == 0)
        def _init_tail():
            pltpu.sync_copy(zeros.at[pl.ds(_RPT * _NS, _RTAIL)],
                            acc.at[pl.ds(_RPT * _NS, _RTAIL)])

        plsc.subcore_barrier()
        base = wid * _EPW

        def chunk(c, carry):
            off = base + c * _K
            pltpu.sync_copy(dstids.at[pl.ds(off, _K)], idx_v)
            pltpu.sync_copy(eh.at[pl.ds(off, _K)], rows_v)
            pltpu.sync_copy(rows_v, acc.at[idx_v], add=True)
            return carry

        lax.fori_loop(0, _NCH, chunk, 0)
        plsc.subcore_barrier()
        pltpu.sync_copy(acc.at[pl.ds(sid * _RPT, _RPT)],
                        out.at[pl.ds(cid * _N + sid * _RPT, _RPT)])

        @pl.when(sid == 0)
        def _out_tail():
            pltpu.sync_copy(acc.at[pl.ds(_RPT * _NS, _RTAIL)],
                            out.at[pl.ds(cid * _N + _RPT * _NS, _RTAIL)])

    return _sc_scatter


def _vec(x):
    return x.reshape(1, -1)


def _w12(layers, w1):
    out = []
    for i, l in enumerate(layers):
        w = w1 if i == 0 else l['W']
        out += [w, _vec(l['b']), _vec(l['g']), _vec(l['be'])]
    return out


def kernel(X, edge_index, edge_attr, params):
    src = edge_index[0]
    dst = edge_index[1]
    ea = jnp.pad(edge_attr, ((0, 0), (0, 1)))

    ne = params['node_enc']
    ee = params['edge_enc']
    proc = params['proc']
    dec = params['dec']

    e_w1 = [s['edge'][0]['W'] for s in proc]
    n_w1 = [s['node'][0]['W'] for s in proc]
    ee_w1 = jnp.pad(ee[0]['W'], ((0, 1), (0, 0)))

    zeros = jnp.zeros((_N, _D), jnp.float32)

    Xh, P, Q = _rowwise_call(
        _node_enc_body, _N, 1000, [X],
        _w12(ne, ne[0]['W'][0:1, :]) + [e_w1[0][0:_D], e_w1[0][_D:2 * _D]],
        [(_N, _D)] * 3)

    Eh = _rowwise_call(_edge_enc_body, _E, 2000, [ea], _w12(ee, ee_w1),
                       [(_E, _D)])

    for s in range(2):
        Gs, Gd = _sc_gather_kernel()(src, dst, P, Q)
        Eh = _rowwise_call(
            _edge_mlp_body, _E, 2000, [Gs, Gd, Eh],
            _w12(proc[s]['edge'], e_w1[s][2 * _D:3 * _D]), [(_E, _D)])
        parts = _sc_scatter_kernel()(Eh, dst, zeros)
        nw = _w12(proc[s]['node'], n_w1[s][0:_D]) + [n_w1[s][_D:2 * _D]]
        nw = [nw[0], nw[12]] + nw[1:12]
        if s == 0:
            Xh, P, Q = _rowwise_call(
                _node_mlp_body, _N, 1000,
                [Xh, parts[:_N], parts[_N:]],
                nw + [e_w1[1][0:_D], e_w1[1][_D:2 * _D]], [(_N, _D)] * 3)
        else:
            dw = [dec[0]['W'], _vec(dec[0]['b']), _vec(dec[0]['g']),
                  _vec(dec[0]['be']),
                  dec[1]['W'], _vec(dec[1]['b']), _vec(dec[1]['g']),
                  _vec(dec[1]['be']),
                  jnp.pad(dec[2]['W'], ((0, 0), (0, _D - 6))),
                  jnp.pad(_vec(dec[2]['b']), ((0, 0), (0, _D - 6)))]
            Ypad = _rowwise_call(_node_dec_body, _N, 1000,
                                 [Xh, parts[:_N], parts[_N:]], nw + dw,
                                 [(_N, _D)])
    return Ypad[:, :6]


# speedup vs baseline: 3.8242x; 1.3244x over previous
import functools

import jax
import jax.numpy as jnp
from jax import lax
from jax.experimental import pallas as pl
from jax.experimental.pallas import tpu as pltpu
from jax.experimental.pallas import tpu_sc as plsc

_N = 10000
_E = 320000
_D = 128
_EPS = 1e-5

_NC = 2
_NS = 16
_NW = _NC * _NS
_EPW = _E // _NW
_K = 40
_NCH = _EPW // _K
_RPT = 624
_RTAIL = _N - _RPT * _NS


def _ln(x, g, b):
    mu = jnp.mean(x, axis=-1, keepdims=True)
    xc = x - mu
    var = jnp.mean(xc * xc, axis=-1, keepdims=True)
    return xc * lax.rsqrt(var + _EPS) * g + b


def _dot(a, w):
    return jnp.dot(a, w, preferred_element_type=jnp.float32)


def _block(h, g, e, relu=True):
    if relu:
        h = jnp.maximum(h, 0.0)
    return _ln(h, g, e)


def _tail2(h, w2, b2, g2, e2, w3, b3, g3, e3):
    h = _block(_dot(h, w2[...]) + b2[...], g2[...], e2[...])
    return _block(_dot(h, w3[...]) + b3[...], g3[...], e3[...], relu=False)


def _edge_enc_body(x_ref, w1, b1, g1, e1, w2, b2, g2, e2, w3, b3, g3, e3,
                   o_ref):
    h = _block(_dot(x_ref[...], w1[...]) + b1[...], g1[...], e1[...])
    o_ref[...] = _tail2(h, w2, b2, g2, e2, w3, b3, g3, e3)


def _node_enc_body(x_ref, w1r, b1, g1, e1, w2, b2, g2, e2, w3, b3, g3, e3,
                   wpa, wpb, xh_ref, p_ref, q_ref):
    x0 = x_ref[...][:, 0:1]
    h = _block(x0 * w1r[...] + b1[...], g1[...], e1[...])
    xh = _tail2(h, w2, b2, g2, e2, w3, b3, g3, e3)
    xh_ref[...] = xh
    p_ref[...] = _dot(xh, wpa[...])
    q_ref[...] = _dot(xh, wpb[...])


def _edge_mlp_body(g_ref, eh_ref, w1c, b1, g1, e1, w2, b2, g2, e2,
                   w3, b3, g3, e3, o_ref):
    eh = eh_ref[...]
    h = _block(g_ref[...] + _dot(eh, w1c[...]) + b1[...],
               g1[...], e1[...])
    o_ref[...] = eh + _tail2(h, w2, b2, g2, e2, w3, b3, g3, e3)


def _node_mlp_body(xh_ref, aa_ref, ab_ref, w1a, w1b, b1, g1, e1, w2, b2, g2,
                   e2, w3, b3, g3, e3, wpa, wpb, xh_ref_o, p_ref, q_ref):
    xh = xh_ref[...]
    agg = aa_ref[...] + ab_ref[...]
    h = _block(_dot(xh, w1a[...]) + _dot(agg, w1b[...]) + b1[...],
               g1[...], e1[...])
    xh2 = xh + _tail2(h, w2, b2, g2, e2, w3, b3, g3, e3)
    xh_ref_o[...] = xh2
    p_ref[...] = _dot(xh2, wpa[...])
    q_ref[...] = _dot(xh2, wpb[...])


def _node_dec_body(xh_ref, aa_ref, ab_ref, w1a, w1b, b1, g1, e1, w2, b2, g2,
                   e2, w3, b3, g3, e3, dw1, db1, dg1, de1, dw2, db2, dg2, de2,
                   dw3, db3, y_ref):
    xh = xh_ref[...]
    agg = aa_ref[...] + ab_ref[...]
    h = _block(_dot(xh, w1a[...]) + _dot(agg, w1b[...]) + b1[...],
               g1[...], e1[...])
    xh2 = xh + _tail2(h, w2, b2, g2, e2, w3, b3, g3, e3)
    h = _block(_dot(xh2, dw1[...]) + db1[...], dg1[...], de1[...])
    h = _block(_dot(h, dw2[...]) + db2[...], dg2[...], de2[...])
    y_ref[...] = _dot(h, dw3[...]) + db3[...]


def _rowwise_call(body, nrows, rblk, data, weights, out_shapes):
    grid = (nrows // rblk,)
    in_specs = (
        [pl.BlockSpec((rblk, a.shape[1]), lambda i: (i, 0)) for a in data]
        + [pl.BlockSpec(w.shape, functools.partial(lambda nd, i: (0,) * nd,
                                                   w.ndim))
           for w in weights])
    out_specs = [pl.BlockSpec((rblk, s[1]), lambda i: (i, 0))
                 for s in out_shapes]
    out_shape = [jax.ShapeDtypeStruct(s, jnp.float32) for s in out_shapes]
    if len(out_shapes) == 1:
        out_specs, out_shape = out_specs[0], out_shape[0]
    return pl.pallas_call(
        body, grid=grid, in_specs=in_specs, out_specs=out_specs,
        out_shape=out_shape)(*data, *weights)


_NB = 5
_NBS = 2


@functools.cache
def _sc_gather_kernel():
    mesh = plsc.VectorSubcoreMesh(core_axis_name="c", subcore_axis_name="s")

    @functools.partial(
        pl.kernel,
        mesh=mesh,
        out_type=jax.ShapeDtypeStruct((_E, _D), jnp.float32),
        scratch_types=[pltpu.VMEM((_NCH, _K), jnp.int32),
                       pltpu.VMEM((_NCH, _K), jnp.int32),
                       [pltpu.VMEM((_K, _D), jnp.float32)
                        for _ in range(_NB)],
                       [pltpu.VMEM((_K, _D), jnp.float32)
                        for _ in range(_NB)],
                       pltpu.SemaphoreType.DMA,
                       pltpu.SemaphoreType.DMA],
    )
    def _sc_gather(src3d, dst3d, p, q, g, srcv, dstv, bps, bqs, gsem, ssem):
        wid = lax.axis_index("s") * _NC + lax.axis_index("c")
        base = wid * _EPW
        pltpu.sync_copy(src3d.at[wid], srcv)
        pltpu.sync_copy(dst3d.at[wid], dstv)

        def outer(o, carry):
            c0 = o * _NB
            gd = []
            for b in range(_NB):
                c = c0 + b
                gd.append(pltpu.async_copy(p.at[srcv.at[c]], bps[b], gsem))
                gd.append(pltpu.async_copy(q.at[dstv.at[c]], bqs[b], gsem))
            sd = []
            for b in range(_NB):
                c = c0 + b
                gd[2 * b].wait()
                gd[2 * b + 1].wait()
                bp, bq = bps[b], bqs[b]

                def add_row(r, carry2, bp=bp, bq=bq):
                    for j in range(_D // 16):
                        s = pl.ds(j * 16, 16)
                        bp[r, s] = bp[r, s] + bq[r, s]
                    return carry2

                lax.fori_loop(0, _K, add_row, 0)
                sd.append(pltpu.async_copy(
                    bp, g.at[pl.ds(base + c * _K, _K)], ssem))
            for d in sd:
                d.wait()
            return carry

        lax.fori_loop(0, _NCH // _NB, outer, 0)

    return _sc_gather


@functools.cache
def _sc_scatter_kernel():
    mesh = plsc.VectorSubcoreMesh(core_axis_name="c", subcore_axis_name="s")

    @functools.partial(
        pl.kernel,
        mesh=mesh,
        out_type=jax.ShapeDtypeStruct((2 * _N, _D), jnp.float32),
        scratch_types=[pltpu.VMEM((_NCH, _K), jnp.int32),
                       [pltpu.VMEM((_K, _D), jnp.float32)
                        for _ in range(_NBS)],
                       pltpu.SemaphoreType.DMA,
                       pltpu.VMEM_SHARED((_N, _D), jnp.float32)],
    )
    def _sc_scatter(eh, dst3d, zeros, out, dstv, bufs, lsem, acc):
        cid = lax.axis_index("c")
        sid = lax.axis_index("s")
        wid = sid * _NC + cid
        pltpu.sync_copy(zeros.at[pl.ds(sid * _RPT, _RPT)],
                        acc.at[pl.ds(sid * _RPT, _RPT)])

        @pl.when(sid == 0)
        def _init_tail():
            pltpu.sync_copy(zeros.at[pl.ds(_RPT * _NS, _RTAIL)],
                            acc.at[pl.ds(_RPT * _NS, _RTAIL)])

        pltpu.sync_copy(dst3d.at[wid], dstv)
        plsc.subcore_barrier()
        base = wid * _EPW

        def outer(o, carry):
            c0 = o * _NBS
            ld = []
            for b in range(_NBS):
                c = c0 + b
                ld.append(pltpu.async_copy(
                    eh.at[pl.ds(base + c * _K, _K)], bufs[b], lsem))
            for b in range(_NBS):
                c = c0 + b
                ld[b].wait()
                pltpu.sync_copy(bufs[b], acc.at[dstv.at[c]], add=True)
            return carry

        lax.fori_loop(0, _NCH // _NBS, outer, 0)
        plsc.subcore_barrier()
        pltpu.sync_copy(acc.at[pl.ds(sid * _RPT, _RPT)],
                        out.at[pl.ds(cid * _N + sid * _RPT, _RPT)])

        @pl.when(sid == 0)
        def _out_tail():
            pltpu.sync_copy(acc.at[pl.ds(_RPT * _NS, _RTAIL)],
                            out.at[pl.ds(cid * _N + _RPT * _NS, _RTAIL)])

    return _sc_scatter


def _vec(x):
    return x.reshape(1, -1)


def _w12(layers, w1):
    out = []
    for i, l in enumerate(layers):
        w = w1 if i == 0 else l['W']
        out += [w, _vec(l['b']), _vec(l['g']), _vec(l['be'])]
    return out


def kernel(X, edge_index, edge_attr, params):
    src3d = edge_index[0].reshape(_NW, _NCH, _K)
    dst3d = edge_index[1].reshape(_NW, _NCH, _K)
    ea = jnp.pad(edge_attr, ((0, 0), (0, 1)))

    ne = params['node_enc']
    ee = params['edge_enc']
    proc = params['proc']
    dec = params['dec']

    e_w1 = [s['edge'][0]['W'] for s in proc]
    n_w1 = [s['node'][0]['W'] for s in proc]
    ee_w1 = jnp.pad(ee[0]['W'], ((0, 1), (0, 0)))

    zeros = jnp.zeros((_N, _D), jnp.float32)

    Xh, P, Q = _rowwise_call(
        _node_enc_body, _N, 1000, [X],
        _w12(ne, ne[0]['W'][0:1, :]) + [e_w1[0][0:_D], e_w1[0][_D:2 * _D]],
        [(_N, _D)] * 3)

    Eh = _rowwise_call(_edge_enc_body, _E, 2000, [ea], _w12(ee, ee_w1),
                       [(_E, _D)])

    for s in range(2):
        G = _sc_gather_kernel()(src3d, dst3d, P, Q)
        Eh = _rowwise_call(
            _edge_mlp_body, _E, 2000, [G, Eh],
            _w12(proc[s]['edge'], e_w1[s][2 * _D:3 * _D]), [(_E, _D)])
        parts = _sc_scatter_kernel()(Eh, dst3d, zeros)
        nw = _w12(proc[s]['node'], n_w1[s][0:_D]) + [n_w1[s][_D:2 * _D]]
        nw = [nw[0], nw[12]] + nw[1:12]
        if s == 0:
            Xh, P, Q = _rowwise_call(
                _node_mlp_body, _N, 1000,
                [Xh, parts[:_N], parts[_N:]],
                nw + [e_w1[1][0:_D], e_w1[1][_D:2 * _D]], [(_N, _D)] * 3)
        else:
            dw = [dec[0]['W'], _vec(dec[0]['b']), _vec(dec[0]['g']),
                  _vec(dec[0]['be']),
                  dec[1]['W'], _vec(dec[1]['b']), _vec(dec[1]['g']),
                  _vec(dec[1]['be']),
                  jnp.pad(dec[2]['W'], ((0, 0), (0, _D - 6))),
                  jnp.pad(_vec(dec[2]['b']), ((0, 0), (0, _D - 6)))]
            Ypad = _rowwise_call(_node_dec_body, _N, 1000,
                                 [Xh, parts[:_N], parts[_N:]], nw + dw,
                                 [(_N, _D)])
    return Ypad[:, :6]


# speedup vs baseline: 3.8636x; 1.0103x over previous
import functools

import jax
import jax.numpy as jnp
from jax import lax
from jax.experimental import pallas as pl
from jax.experimental.pallas import tpu as pltpu
from jax.experimental.pallas import tpu_sc as plsc

_N = 10000
_E = 320000
_D = 128
_EPS = 1e-5

_NC = 2
_NS = 16
_NW = _NC * _NS
_EPW = _E // _NW
_K = 40
_NCH = _EPW // _K
_RPT = 624
_RTAIL = _N - _RPT * _NS


def _ln(x, g, b):
    mu = jnp.mean(x, axis=-1, keepdims=True)
    xc = x - mu
    var = jnp.mean(xc * xc, axis=-1, keepdims=True)
    return xc * lax.rsqrt(var + _EPS) * g + b


def _dot(a, w):
    return jnp.dot(a, w, preferred_element_type=jnp.float32)


def _block(h, g, e, relu=True):
    if relu:
        h = jnp.maximum(h, 0.0)
    return _ln(h, g, e)


def _tail2(h, w2, b2, g2, e2, w3, b3, g3, e3):
    h = _block(_dot(h, w2[...]) + b2[...], g2[...], e2[...])
    return _block(_dot(h, w3[...]) + b3[...], g3[...], e3[...], relu=False)


def _edge_enc_body(x_ref, w1, b1, g1, e1, w2, b2, g2, e2, w3, b3, g3, e3,
                   o_ref):
    h = _block(_dot(x_ref[...], w1[...]) + b1[...], g1[...], e1[...])
    o_ref[...] = _tail2(h, w2, b2, g2, e2, w3, b3, g3, e3)


def _node_enc_body(x_ref, w1r, b1, g1, e1, w2, b2, g2, e2, w3, b3, g3, e3,
                   wpa, wpb, xh_ref, p_ref, q_ref):
    x0 = x_ref[...][:, 0:1]
    h = _block(x0 * w1r[...] + b1[...], g1[...], e1[...])
    xh = _tail2(h, w2, b2, g2, e2, w3, b3, g3, e3)
    xh_ref[...] = xh
    p_ref[...] = _dot(xh, wpa[...])
    q_ref[...] = _dot(xh, wpb[...])


def _edge_enc_mlp_body(x_ref, g_ref,
                       ew1, eb1, eg1, ee1, ew2, eb2, eg2, ee2, ew3, eb3, eg3,
                       ee3, w1c, b1, g1, e1, w2, b2, g2, e2, w3, b3, g3, e3,
                       o_ref):
    eh = _block(_dot(x_ref[...], ew1[...]) + eb1[...], eg1[...], ee1[...])
    eh = _tail2(eh, ew2, eb2, eg2, ee2, ew3, eb3, eg3, ee3)
    h = _block(g_ref[...] + _dot(eh, w1c[...]) + b1[...], g1[...], e1[...])
    o_ref[...] = eh + _tail2(h, w2, b2, g2, e2, w3, b3, g3, e3)


def _edge_mlp_body(g_ref, eh_ref, w1c, b1, g1, e1, w2, b2, g2, e2,
                   w3, b3, g3, e3, o_ref):
    eh = eh_ref[...]
    h = _block(g_ref[...] + _dot(eh, w1c[...]) + b1[...],
               g1[...], e1[...])
    o_ref[...] = eh + _tail2(h, w2, b2, g2, e2, w3, b3, g3, e3)


def _node_mlp_body(xh_ref, aa_ref, ab_ref, w1a, w1b, b1, g1, e1, w2, b2, g2,
                   e2, w3, b3, g3, e3, wpa, wpb, xh_ref_o, p_ref, q_ref):
    xh = xh_ref[...]
    agg = aa_ref[...] + ab_ref[...]
    h = _block(_dot(xh, w1a[...]) + _dot(agg, w1b[...]) + b1[...],
               g1[...], e1[...])
    xh2 = xh + _tail2(h, w2, b2, g2, e2, w3, b3, g3, e3)
    xh_ref_o[...] = xh2
    p_ref[...] = _dot(xh2, wpa[...])
    q_ref[...] = _dot(xh2, wpb[...])


def _node_dec_body(xh_ref, aa_ref, ab_ref, w1a, w1b, b1, g1, e1, w2, b2, g2,
                   e2, w3, b3, g3, e3, dw1, db1, dg1, de1, dw2, db2, dg2, de2,
                   dw3, db3, y_ref):
    xh = xh_ref[...]
    agg = aa_ref[...] + ab_ref[...]
    h = _block(_dot(xh, w1a[...]) + _dot(agg, w1b[...]) + b1[...],
               g1[...], e1[...])
    xh2 = xh + _tail2(h, w2, b2, g2, e2, w3, b3, g3, e3)
    h = _block(_dot(xh2, dw1[...]) + db1[...], dg1[...], de1[...])
    h = _block(_dot(h, dw2[...]) + db2[...], dg2[...], de2[...])
    y_ref[...] = _dot(h, dw3[...]) + db3[...]


def _rowwise_call(body, nrows, rblk, data, weights, out_shapes):
    grid = (nrows // rblk,)
    in_specs = (
        [pl.BlockSpec((rblk, a.shape[1]), lambda i: (i, 0)) for a in data]
        + [pl.BlockSpec(w.shape, functools.partial(lambda nd, i: (0,) * nd,
                                                   w.ndim))
           for w in weights])
    out_specs = [pl.BlockSpec((rblk, s[1]), lambda i: (i, 0))
                 for s in out_shapes]
    out_shape = [jax.ShapeDtypeStruct(s, jnp.float32) for s in out_shapes]
    if len(out_shapes) == 1:
        out_specs, out_shape = out_specs[0], out_shape[0]
    return pl.pallas_call(
        body, grid=grid, in_specs=in_specs, out_specs=out_specs,
        out_shape=out_shape)(*data, *weights)


_NB = 5
_NBS = 2


@functools.cache
def _sc_gather_kernel():
    mesh = plsc.VectorSubcoreMesh(core_axis_name="c", subcore_axis_name="s")

    @functools.partial(
        pl.kernel,
        mesh=mesh,
        out_type=jax.ShapeDtypeStruct((_E, _D), jnp.float32),
        scratch_types=[pltpu.VMEM((_NCH, _K), jnp.int32),
                       pltpu.VMEM((_NCH, _K), jnp.int32),
                       [pltpu.VMEM((_K, _D), jnp.float32)
                        for _ in range(_NB)],
                       [pltpu.VMEM((_K, _D), jnp.float32)
                        for _ in range(_NB)],
                       pltpu.SemaphoreType.DMA,
                       pltpu.SemaphoreType.DMA],
    )
    def _sc_gather(src3d, dst3d, p, q, g, srcv, dstv, bps, bqs, gsem, ssem):
        wid = lax.axis_index("s") * _NC + lax.axis_index("c")
        base = wid * _EPW
        pltpu.sync_copy(src3d.at[wid], srcv)
        pltpu.sync_copy(dst3d.at[wid], dstv)

        def outer(o, carry):
            c0 = o * _NB
            gd = []
            for b in range(_NB):
                c = c0 + b
                gd.append(pltpu.async_copy(p.at[srcv.at[c]], bps[b], gsem))
                gd.append(pltpu.async_copy(q.at[dstv.at[c]], bqs[b], gsem))
            sd = []
            for b in range(_NB):
                c = c0 + b
                gd[2 * b].wait()
                gd[2 * b + 1].wait()
                bp, bq = bps[b], bqs[b]

                def add_row(r, carry2, bp=bp, bq=bq):
                    for j in range(_D // 16):
                        s = pl.ds(j * 16, 16)
                        bp[r, s] = bp[r, s] + bq[r, s]
                    return carry2

                lax.fori_loop(0, _K, add_row, 0)
                sd.append(pltpu.async_copy(
                    bp, g.at[pl.ds(base + c * _K, _K)], ssem))
            for d in sd:
                d.wait()
            return carry

        lax.fori_loop(0, _NCH // _NB, outer, 0)

    return _sc_gather


@functools.cache
def _sc_scatter_kernel():
    mesh = plsc.VectorSubcoreMesh(core_axis_name="c", subcore_axis_name="s")

    @functools.partial(
        pl.kernel,
        mesh=mesh,
        out_type=jax.ShapeDtypeStruct((2 * _N, _D), jnp.float32),
        scratch_types=[pltpu.VMEM((_NCH, _K), jnp.int32),
                       [pltpu.VMEM((_K, _D), jnp.float32)
                        for _ in range(_NBS)],
                       pltpu.SemaphoreType.DMA,
                       pltpu.VMEM_SHARED((_N, _D), jnp.float32)],
    )
    def _sc_scatter(eh, dst3d, zeros, out, dstv, bufs, lsem, acc):
        cid = lax.axis_index("c")
        sid = lax.axis_index("s")
        wid = sid * _NC + cid
        pltpu.sync_copy(zeros.at[pl.ds(sid * _RPT, _RPT)],
                        acc.at[pl.ds(sid * _RPT, _RPT)])

        @pl.when(sid == 0)
        def _init_tail():
            pltpu.sync_copy(zeros.at[pl.ds(_RPT * _NS, _RTAIL)],
                            acc.at[pl.ds(_RPT * _NS, _RTAIL)])

        pltpu.sync_copy(dst3d.at[wid], dstv)
        plsc.subcore_barrier()
        base = wid * _EPW

        def outer(o, carry):
            c0 = o * _NBS
            ld = []
            for b in range(_NBS):
                c = c0 + b
                ld.append(pltpu.async_copy(
                    eh.at[pl.ds(base + c * _K, _K)], bufs[b], lsem))
            for b in range(_NBS):
                c = c0 + b
                ld[b].wait()
                pltpu.sync_copy(bufs[b], acc.at[dstv.at[c]], add=True)
            return carry

        lax.fori_loop(0, _NCH // _NBS, outer, 0)
        plsc.subcore_barrier()
        pltpu.sync_copy(acc.at[pl.ds(sid * _RPT, _RPT)],
                        out.at[pl.ds(cid * _N + sid * _RPT, _RPT)])

        @pl.when(sid == 0)
        def _out_tail():
            pltpu.sync_copy(acc.at[pl.ds(_RPT * _NS, _RTAIL)],
                            out.at[pl.ds(cid * _N + _RPT * _NS, _RTAIL)])

    return _sc_scatter


def _vec(x):
    return x.reshape(1, -1)


def _w12(layers, w1):
    out = []
    for i, l in enumerate(layers):
        w = w1 if i == 0 else l['W']
        out += [w, _vec(l['b']), _vec(l['g']), _vec(l['be'])]
    return out


def kernel(X, edge_index, edge_attr, params):
    src3d = edge_index[0].reshape(_NW, _NCH, _K)
    dst3d = edge_index[1].reshape(_NW, _NCH, _K)
    ea = jnp.pad(edge_attr, ((0, 0), (0, 1)))

    ne = params['node_enc']
    ee = params['edge_enc']
    proc = params['proc']
    dec = params['dec']

    e_w1 = [s['edge'][0]['W'] for s in proc]
    n_w1 = [s['node'][0]['W'] for s in proc]
    ee_w1 = jnp.pad(ee[0]['W'], ((0, 1), (0, 0)))

    zeros = jnp.zeros((_N, _D), jnp.float32)

    Xh, P, Q = _rowwise_call(
        _node_enc_body, _N, 1000, [X],
        _w12(ne, ne[0]['W'][0:1, :]) + [e_w1[0][0:_D], e_w1[0][_D:2 * _D]],
        [(_N, _D)] * 3)

    for s in range(2):
        G = _sc_gather_kernel()(src3d, dst3d, P, Q)
        if s == 0:
            Eh = _rowwise_call(
                _edge_enc_mlp_body, _E, 4000, [ea, G],
                _w12(ee, ee_w1) + _w12(proc[0]['edge'],
                                       e_w1[0][2 * _D:3 * _D]),
                [(_E, _D)])
        else:
            Eh = _rowwise_call(
                _edge_mlp_body, _E, 4000, [G, Eh],
                _w12(proc[s]['edge'], e_w1[s][2 * _D:3 * _D]), [(_E, _D)])
        parts = _sc_scatter_kernel()(Eh, dst3d, zeros)
        nw = _w12(proc[s]['node'], n_w1[s][0:_D]) + [n_w1[s][_D:2 * _D]]
        nw = [nw[0], nw[12]] + nw[1:12]
        if s == 0:
            Xh, P, Q = _rowwise_call(
                _node_mlp_body, _N, 1000,
                [Xh, parts[:_N], parts[_N:]],
                nw + [e_w1[1][0:_D], e_w1[1][_D:2 * _D]], [(_N, _D)] * 3)
        else:
            dw = [dec[0]['W'], _vec(dec[0]['b']), _vec(dec[0]['g']),
                  _vec(dec[0]['be']),
                  dec[1]['W'], _vec(dec[1]['b']), _vec(dec[1]['g']),
                  _vec(dec[1]['be']),
                  jnp.pad(dec[2]['W'], ((0, 0), (0, _D - 6))),
                  jnp.pad(_vec(dec[2]['b']), ((0, 0), (0, _D - 6)))]
            Ypad = _rowwise_call(_node_dec_body, _N, 1000,
                                 [Xh, parts[:_N], parts[_N:]], nw + dw,
                                 [(_N, _D)])
    return Ypad[:, :6]


# speedup vs baseline: 4.6582x; 1.2056x over previous
import functools

import jax
import jax.numpy as jnp
from jax import lax
from jax.experimental import pallas as pl
from jax.experimental.pallas import tpu as pltpu
from jax.experimental.pallas import tpu_sc as plsc

_N = 10000
_E = 320000
_D = 128
_EPS = 1e-5

_NC = 2
_NS = 16
_NW = _NC * _NS
_EH = _E // 2
_EPW = _EH // _NW
_K = 40
_NCH = _EPW // _K
_RPT = 624
_RTAIL = _N - _RPT * _NS
_NB = 5


def _ln(x, g, b):
    mu = jnp.mean(x, axis=-1, keepdims=True)
    xc = x - mu
    var = jnp.mean(xc * xc, axis=-1, keepdims=True)
    return xc * lax.rsqrt(var + _EPS) * g + b


def _dot(a, w):
    return jnp.dot(a, w, preferred_element_type=jnp.float32)


def _block(h, g, e, relu=True):
    if relu:
        h = jnp.maximum(h, 0.0)
    return _ln(h, g, e)


def _tail2(h, w2, b2, g2, e2, w3, b3, g3, e3):
    h = _block(_dot(h, w2[...]) + b2[...], g2[...], e2[...])
    return _block(_dot(h, w3[...]) + b3[...], g3[...], e3[...], relu=False)


def _edge_enc_body(x_ref, w1, b1, g1, e1, w2, b2, g2, e2, w3, b3, g3, e3,
                   o_ref):
    h = _block(_dot(x_ref[...], w1[...]) + b1[...], g1[...], e1[...])
    o_ref[...] = _tail2(h, w2, b2, g2, e2, w3, b3, g3, e3)


def _node_enc_body(x_ref, w1r, b1, g1, e1, w2, b2, g2, e2, w3, b3, g3, e3,
                   wpa, wpb, xh_ref, p_ref, q_ref):
    x0 = x_ref[...][:, 0:1]
    h = _block(x0 * w1r[...] + b1[...], g1[...], e1[...])
    xh = _tail2(h, w2, b2, g2, e2, w3, b3, g3, e3)
    xh_ref[...] = xh
    p_ref[...] = _dot(xh, wpa[...])
    q_ref[...] = _dot(xh, wpb[...])


def _edge_enc_mlp_body(x_ref, g_ref,
                       ew1, eb1, eg1, ee1, ew2, eb2, eg2, ee2, ew3, eb3, eg3,
                       ee3, w1c, b1, g1, e1, w2, b2, g2, e2, w3, b3, g3, e3,
                       o_ref):
    eh = _block(_dot(x_ref[...], ew1[...]) + eb1[...], eg1[...], ee1[...])
    eh = _tail2(eh, ew2, eb2, eg2, ee2, ew3, eb3, eg3, ee3)
    h = _block(g_ref[...] + _dot(eh, w1c[...]) + b1[...], g1[...], e1[...])
    o_ref[...] = eh + _tail2(h, w2, b2, g2, e2, w3, b3, g3, e3)


def _edge_mlp_body(g_ref, eh_ref, w1c, b1, g1, e1, w2, b2, g2, e2,
                   w3, b3, g3, e3, o_ref):
    eh = eh_ref[...]
    h = _block(g_ref[...] + _dot(eh, w1c[...]) + b1[...],
               g1[...], e1[...])
    o_ref[...] = eh + _tail2(h, w2, b2, g2, e2, w3, b3, g3, e3)


def _node_mlp_body(xh_ref, aa_ref, ab_ref, ba_ref, bb_ref, w1a, w1b, b1, g1,
                   e1, w2, b2, g2, e2, w3, b3, g3, e3, wpa, wpb, xh_ref_o,
                   p_ref, q_ref):
    xh = xh_ref[...]
    agg = ((aa_ref[...] + ab_ref[...]) + (ba_ref[...] + bb_ref[...]))
    h = _block(_dot(xh, w1a[...]) + _dot(agg, w1b[...]) + b1[...],
               g1[...], e1[...])
    xh2 = xh + _tail2(h, w2, b2, g2, e2, w3, b3, g3, e3)
    xh_ref_o[...] = xh2
    p_ref[...] = _dot(xh2, wpa[...])
    q_ref[...] = _dot(xh2, wpb[...])


def _node_dec_body(xh_ref, aa_ref, ab_ref, ba_ref, bb_ref, w1a, w1b, b1, g1,
                   e1, w2, b2, g2, e2, w3, b3, g3, e3, dw1, db1, dg1, de1,
                   dw2, db2, dg2, de2, dw3, db3, y_ref):
    xh = xh_ref[...]
    agg = ((aa_ref[...] + ab_ref[...]) + (ba_ref[...] + bb_ref[...]))
    h = _block(_dot(xh, w1a[...]) + _dot(agg, w1b[...]) + b1[...],
               g1[...], e1[...])
    xh2 = xh + _tail2(h, w2, b2, g2, e2, w3, b3, g3, e3)
    h = _block(_dot(xh2, dw1[...]) + db1[...], dg1[...], de1[...])
    h = _block(_dot(h, dw2[...]) + db2[...], dg2[...], de2[...])
    y_ref[...] = _dot(h, dw3[...]) + db3[...]


def _rowwise_call(body, nrows, rblk, data, weights, out_shapes):
    grid = (nrows // rblk,)
    in_specs = (
        [pl.BlockSpec((rblk, a.shape[1]), lambda i: (i, 0)) for a in data]
        + [pl.BlockSpec(w.shape, functools.partial(lambda nd, i: (0,) * nd,
                                                   w.ndim))
           for w in weights])
    out_specs = [pl.BlockSpec((rblk, s[1]), lambda i: (i, 0))
                 for s in out_shapes]
    out_shape = [jax.ShapeDtypeStruct(s, jnp.float32) for s in out_shapes]
    if len(out_shapes) == 1:
        out_specs, out_shape = out_specs[0], out_shape[0]
    return pl.pallas_call(
        body, grid=grid, in_specs=in_specs, out_specs=out_specs,
        out_shape=out_shape)(*data, *weights)


@functools.cache
def _sc_gather_kernel():
    mesh = plsc.VectorSubcoreMesh(core_axis_name="c", subcore_axis_name="s")

    @functools.partial(
        pl.kernel,
        mesh=mesh,
        out_type=jax.ShapeDtypeStruct((_EH, _D), jnp.float32),
        scratch_types=[pltpu.VMEM((_NCH, _K), jnp.int32),
                       pltpu.VMEM((_NCH, _K), jnp.int32),
                       [pltpu.VMEM((_K, _D), jnp.float32)
                        for _ in range(_NB)],
                       [pltpu.VMEM((_K, _D), jnp.float32)
                        for _ in range(_NB)],
                       pltpu.SemaphoreType.DMA,
                       pltpu.SemaphoreType.DMA],
    )
    def _sc_gather(src3d, dst3d, p, q, g, srcv, dstv, bps, bqs, gsem, ssem):
        wid = lax.axis_index("s") * _NC + lax.axis_index("c")
        base = wid * _EPW
        pltpu.sync_copy(src3d.at[wid], srcv)
        pltpu.sync_copy(dst3d.at[wid], dstv)

        def outer(o, carry):
            c0 = o * _NB
            gd = []
            for b in range(_NB):
                c = c0 + b
                gd.append(pltpu.async_copy(p.at[srcv.at[c]], bps[b], gsem))
                gd.append(pltpu.async_copy(q.at[dstv.at[c]], bqs[b], gsem))
            sd = []
            for b in range(_NB):
                c = c0 + b
                gd[2 * b].wait()
                gd[2 * b + 1].wait()
                bp, bq = bps[b], bqs[b]

                def add_row(r, carry2, bp=bp, bq=bq):
                    for j in range(_D // 16):
                        s = pl.ds(j * 16, 16)
                        bp[r, s] = bp[r, s] + bq[r, s]
                    return carry2

                lax.fori_loop(0, _K, add_row, 0)
                sd.append(pltpu.async_copy(
                    bp, g.at[pl.ds(base + c * _K, _K)], ssem))
            for d in sd:
                d.wait()
            return carry

        lax.fori_loop(0, _NCH // _NB, outer, 0)

    return _sc_gather


@functools.cache
def _sc_scatter_kernel():
    mesh = plsc.VectorSubcoreMesh(core_axis_name="c", subcore_axis_name="s")

    @functools.partial(
        pl.kernel,
        mesh=mesh,
        out_type=jax.ShapeDtypeStruct((2 * _N, _D), jnp.float32),
        scratch_types=[pltpu.VMEM((_NCH, _K), jnp.int32),
                       [pltpu.VMEM((_K, _D), jnp.float32)
                        for _ in range(2)],
                       pltpu.SemaphoreType.DMA,
                       pltpu.VMEM_SHARED((_N, _D), jnp.float32)],
    )
    def _sc_scatter(eh, dst3d, zeros, out, dstv, bufs, lsem, acc):
        cid = lax.axis_index("c")
        sid = lax.axis_index("s")
        wid = sid * _NC + cid
        pltpu.sync_copy(zeros.at[pl.ds(sid * _RPT, _RPT)],
                        acc.at[pl.ds(sid * _RPT, _RPT)])

        @pl.when(sid == 0)
        def _init_tail():
            pltpu.sync_copy(zeros.at[pl.ds(_RPT * _NS, _RTAIL)],
                            acc.at[pl.ds(_RPT * _NS, _RTAIL)])

        pltpu.sync_copy(dst3d.at[wid], dstv)
        plsc.subcore_barrier()
        base = wid * _EPW

        def outer(o, carry):
            c0 = o * 2
            ld = []
            for b in range(2):
                c = c0 + b
                ld.append(pltpu.async_copy(
                    eh.at[pl.ds(base + c * _K, _K)], bufs[b], lsem))
            for b in range(2):
                c = c0 + b
                ld[b].wait()
                pltpu.sync_copy(bufs[b], acc.at[dstv.at[c]], add=True)
            return carry

        lax.fori_loop(0, (_NCH - 1) // 2, outer, 0)
        c = _NCH - 1
        pltpu.async_copy(eh.at[pl.ds(base + c * _K, _K)], bufs[0],
                         lsem).wait()
        pltpu.sync_copy(bufs[0], acc.at[dstv.at[c]], add=True)
        plsc.subcore_barrier()
        pltpu.sync_copy(acc.at[pl.ds(sid * _RPT, _RPT)],
                        out.at[pl.ds(cid * _N + sid * _RPT, _RPT)])

        @pl.when(sid == 0)
        def _out_tail():
            pltpu.sync_copy(acc.at[pl.ds(_RPT * _NS, _RTAIL)],
                            out.at[pl.ds(cid * _N + _RPT * _NS, _RTAIL)])

    return _sc_scatter


def _vec(x):
    return x.reshape(1, -1)


def _w12(layers, w1):
    out = []
    for i, l in enumerate(layers):
        w = w1 if i == 0 else l['W']
        out += [w, _vec(l['b']), _vec(l['g']), _vec(l['be'])]
    return out


def kernel(X, edge_index, edge_attr, params):
    ea = jnp.pad(edge_attr, ((0, 0), (0, 1)))
    src3d = [edge_index[0, h * _EH:(h + 1) * _EH].reshape(_NW, _NCH, _K)
             for h in range(2)]
    dst3d = [edge_index[1, h * _EH:(h + 1) * _EH].reshape(_NW, _NCH, _K)
             for h in range(2)]
    eah = [ea[h * _EH:(h + 1) * _EH] for h in range(2)]

    ne = params['node_enc']
    ee = params['edge_enc']
    proc = params['proc']
    dec = params['dec']

    e_w1 = [s['edge'][0]['W'] for s in proc]
    n_w1 = [s['node'][0]['W'] for s in proc]
    ee_w1 = jnp.pad(ee[0]['W'], ((0, 1), (0, 0)))

    zeros = jnp.zeros((_N, _D), jnp.float32)

    Xh, P, Q = _rowwise_call(
        _node_enc_body, _N, 1000, [X],
        _w12(ne, ne[0]['W'][0:1, :]) + [e_w1[0][0:_D], e_w1[0][_D:2 * _D]],
        [(_N, _D)] * 3)

    Eh = [None, None]
    for s in range(2):
        G = [_sc_gather_kernel()(src3d[h], dst3d[h], P, Q) for h in range(2)]
        if s == 0:
            ew = _w12(ee, ee_w1) + _w12(proc[0]['edge'],
                                        e_w1[0][2 * _D:3 * _D])
            Eh = [_rowwise_call(_edge_enc_mlp_body, _EH, 4000,
                                [eah[h], G[h]], ew, [(_EH, _D)])
                  for h in range(2)]
        else:
            ew = _w12(proc[s]['edge'], e_w1[s][2 * _D:3 * _D])
            Eh = [_rowwise_call(_edge_mlp_body, _EH, 4000,
                                [G[h], Eh[h]], ew, [(_EH, _D)])
                  for h in range(2)]
        parts = [_sc_scatter_kernel()(Eh[h], dst3d[h], zeros)
                 for h in range(2)]
        aggs = [parts[0][:_N], parts[0][_N:], parts[1][:_N], parts[1][_N:]]
        nw = _w12(proc[s]['node'], n_w1[s][0:_D]) + [n_w1[s][_D:2 * _D]]
        nw = [nw[0], nw[12]] + nw[1:12]
        if s == 0:
            Xh, P, Q = _rowwise_call(
                _node_mlp_body, _N, 1000, [Xh] + aggs,
                nw + [e_w1[1][0:_D], e_w1[1][_D:2 * _D]], [(_N, _D)] * 3)
        else:
            dw = [dec[0]['W'], _vec(dec[0]['b']), _vec(dec[0]['g']),
                  _vec(dec[0]['be']),
                  dec[1]['W'], _vec(dec[1]['b']), _vec(dec[1]['g']),
                  _vec(dec[1]['be']),
                  jnp.pad(dec[2]['W'], ((0, 0), (0, _D - 6))),
                  jnp.pad(_vec(dec[2]['b']), ((0, 0), (0, _D - 6)))]
            Ypad = _rowwise_call(_node_dec_body, _N, 1000, [Xh] + aggs,
                                 nw + dw, [(_N, _D)])
    return Ypad[:, :6]


# speedup vs baseline: 5.2962x; 1.1370x over previous
import functools

import jax
import jax.numpy as jnp
from jax import lax
from jax.experimental import pallas as pl
from jax.experimental.pallas import tpu as pltpu
from jax.experimental.pallas import tpu_sc as plsc

_N = 10000
_E = 320000
_D = 128
_EPS = 1e-5

_NC = 2
_NS = 16
_NW = _NC * _NS
_EH = _E // 2
_EPW = _EH // _NW
_K = 40
_NCH = _EPW // _K
_RPT = 624
_RTAIL = _N - _RPT * _NS
_NB = 5


def _ln(x, g, b):
    n = x.shape[-1]
    o = jnp.full((n, n), 1.0 / n, jnp.float32)
    mu = _dot(x, o)
    xc = x - mu
    var = _dot(xc * xc, o)
    return xc * lax.rsqrt(var + _EPS) * g + b


def _dot(a, w):
    return jnp.dot(a, w, preferred_element_type=jnp.float32)


def _block(h, g, e, relu=True):
    if relu:
        h = jnp.maximum(h, 0.0)
    return _ln(h, g, e)


def _tail2(h, w2, b2, g2, e2, w3, b3, g3, e3):
    h = _block(_dot(h, w2[...]) + b2[...], g2[...], e2[...])
    return _block(_dot(h, w3[...]) + b3[...], g3[...], e3[...], relu=False)


def _edge_enc_body(x_ref, w1, b1, g1, e1, w2, b2, g2, e2, w3, b3, g3, e3,
                   o_ref):
    h = _block(_dot(x_ref[...], w1[...]) + b1[...], g1[...], e1[...])
    o_ref[...] = _tail2(h, w2, b2, g2, e2, w3, b3, g3, e3)


def _node_enc_body(x_ref, w1r, b1, g1, e1, w2, b2, g2, e2, w3, b3, g3, e3,
                   wpa, wpb, xh_ref, p_ref, q_ref):
    x0 = x_ref[...][:, 0:1]
    h = _block(x0 * w1r[...] + b1[...], g1[...], e1[...])
    xh = _tail2(h, w2, b2, g2, e2, w3, b3, g3, e3)
    xh_ref[...] = xh
    p_ref[...] = _dot(xh, wpa[...])
    q_ref[...] = _dot(xh, wpb[...])


def _edge_enc_mlp_body(x_ref, g_ref,
                       ew1, eb1, eg1, ee1, ew2, eb2, eg2, ee2, ew3, eb3, eg3,
                       ee3, w1c, b1, g1, e1, w2, b2, g2, e2, w3, b3, g3, e3,
                       o_ref):
    eh = _block(_dot(x_ref[...], ew1[...]) + eb1[...], eg1[...], ee1[...])
    eh = _tail2(eh, ew2, eb2, eg2, ee2, ew3, eb3, eg3, ee3)
    h = _block(g_ref[...] + _dot(eh, w1c[...]) + b1[...], g1[...], e1[...])
    o_ref[...] = eh + _tail2(h, w2, b2, g2, e2, w3, b3, g3, e3)


def _edge_mlp_body(g_ref, eh_ref, w1c, b1, g1, e1, w2, b2, g2, e2,
                   w3, b3, g3, e3, o_ref):
    eh = eh_ref[...]
    h = _block(g_ref[...] + _dot(eh, w1c[...]) + b1[...],
               g1[...], e1[...])
    o_ref[...] = eh + _tail2(h, w2, b2, g2, e2, w3, b3, g3, e3)


def _node_mlp_body(xh_ref, aa_ref, ab_ref, ba_ref, bb_ref, w1a, w1b, b1, g1,
                   e1, w2, b2, g2, e2, w3, b3, g3, e3, wpa, wpb, xh_ref_o,
                   p_ref, q_ref):
    xh = xh_ref[...]
    agg = ((aa_ref[...] + ab_ref[...]) + (ba_ref[...] + bb_ref[...]))
    h = _block(_dot(xh, w1a[...]) + _dot(agg, w1b[...]) + b1[...],
               g1[...], e1[...])
    xh2 = xh + _tail2(h, w2, b2, g2, e2, w3, b3, g3, e3)
    xh_ref_o[...] = xh2
    p_ref[...] = _dot(xh2, wpa[...])
    q_ref[...] = _dot(xh2, wpb[...])


def _node_dec_body(xh_ref, aa_ref, ab_ref, ba_ref, bb_ref, w1a, w1b, b1, g1,
                   e1, w2, b2, g2, e2, w3, b3, g3, e3, dw1, db1, dg1, de1,
                   dw2, db2, dg2, de2, dw3, db3, y_ref):
    xh = xh_ref[...]
    agg = ((aa_ref[...] + ab_ref[...]) + (ba_ref[...] + bb_ref[...]))
    h = _block(_dot(xh, w1a[...]) + _dot(agg, w1b[...]) + b1[...],
               g1[...], e1[...])
    xh2 = xh + _tail2(h, w2, b2, g2, e2, w3, b3, g3, e3)
    h = _block(_dot(xh2, dw1[...]) + db1[...], dg1[...], de1[...])
    h = _block(_dot(h, dw2[...]) + db2[...], dg2[...], de2[...])
    y_ref[...] = _dot(h, dw3[...]) + db3[...]


def _rowwise_call(body, nrows, rblk, data, weights, out_shapes):
    grid = (nrows // rblk,)
    in_specs = (
        [pl.BlockSpec((rblk, a.shape[1]), lambda i: (i, 0)) for a in data]
        + [pl.BlockSpec(w.shape, functools.partial(lambda nd, i: (0,) * nd,
                                                   w.ndim))
           for w in weights])
    out_specs = [pl.BlockSpec((rblk, s[1]), lambda i: (i, 0))
                 for s in out_shapes]
    out_shape = [jax.ShapeDtypeStruct(s, jnp.float32) for s in out_shapes]
    if len(out_shapes) == 1:
        out_specs, out_shape = out_specs[0], out_shape[0]
    return pl.pallas_call(
        body, grid=grid, in_specs=in_specs, out_specs=out_specs,
        out_shape=out_shape)(*data, *weights)


@functools.cache
def _sc_gather_kernel():
    mesh = plsc.VectorSubcoreMesh(core_axis_name="c", subcore_axis_name="s")

    @functools.partial(
        pl.kernel,
        mesh=mesh,
        out_type=jax.ShapeDtypeStruct((_EH, _D), jnp.float32),
        scratch_types=[pltpu.VMEM((_NCH, _K), jnp.int32),
                       pltpu.VMEM((_NCH, _K), jnp.int32),
                       [pltpu.VMEM((_K, _D), jnp.float32)
                        for _ in range(_NB)],
                       [pltpu.VMEM((_K, _D), jnp.float32)
                        for _ in range(_NB)],
                       pltpu.SemaphoreType.DMA,
                       pltpu.SemaphoreType.DMA],
    )
    def _sc_gather(src3d, dst3d, p, q, g, srcv, dstv, bps, bqs, gsem, ssem):
        wid = lax.axis_index("s") * _NC + lax.axis_index("c")
        base = wid * _EPW
        pltpu.sync_copy(src3d.at[wid], srcv)
        pltpu.sync_copy(dst3d.at[wid], dstv)

        def outer(o, carry):
            c0 = o * _NB
            gd = []
            for b in range(_NB):
                c = c0 + b
                gd.append(pltpu.async_copy(p.at[srcv.at[c]], bps[b], gsem))
                gd.append(pltpu.async_copy(q.at[dstv.at[c]], bqs[b], gsem))
            sd = []
            for b in range(_NB):
                c = c0 + b
                gd[2 * b].wait()
                gd[2 * b + 1].wait()
                bp, bq = bps[b], bqs[b]

                def add_row(r, carry2, bp=bp, bq=bq):
                    for j in range(_D // 16):
                        s = pl.ds(j * 16, 16)
                        bp[r, s] = bp[r, s] + bq[r, s]
                    return carry2

                lax.fori_loop(0, _K, add_row, 0)
                sd.append(pltpu.async_copy(
                    bp, g.at[pl.ds(base + c * _K, _K)], ssem))
            for d in sd:
                d.wait()
            return carry

        lax.fori_loop(0, _NCH // _NB, outer, 0)

    return _sc_gather


@functools.cache
def _sc_scatter_kernel():
    mesh = plsc.VectorSubcoreMesh(core_axis_name="c", subcore_axis_name="s")

    @functools.partial(
        pl.kernel,
        mesh=mesh,
        out_type=jax.ShapeDtypeStruct((2 * _N, _D), jnp.float32),
        scratch_types=[pltpu.VMEM((_NCH, _K), jnp.int32),
                       [pltpu.VMEM((_K, _D), jnp.float32)
                        for _ in range(2)],
                       pltpu.SemaphoreType.DMA,
                       pltpu.VMEM_SHARED((_N, _D), jnp.float32)],
    )
    def _sc_scatter(eh, dst3d, zeros, out, dstv, bufs, lsem, acc):
        cid = lax.axis_index("c")
        sid = lax.axis_index("s")
        wid = sid * _NC + cid
        pltpu.sync_copy(zeros.at[pl.ds(sid * _RPT, _RPT)],
                        acc.at[pl.ds(sid * _RPT, _RPT)])

        @pl.when(sid == 0)
        def _init_tail():
            pltpu.sync_copy(zeros.at[pl.ds(_RPT * _NS, _RTAIL)],
                            acc.at[pl.ds(_RPT * _NS, _RTAIL)])

        pltpu.sync_copy(dst3d.at[wid], dstv)
        plsc.subcore_barrier()
        base = wid * _EPW

        def outer(o, carry):
            c0 = o * 2
            ld = []
            for b in range(2):
                c = c0 + b
                ld.append(pltpu.async_copy(
                    eh.at[pl.ds(base + c * _K, _K)], bufs[b], lsem))
            for b in range(2):
                c = c0 + b
                ld[b].wait()
                pltpu.sync_copy(bufs[b], acc.at[dstv.at[c]], add=True)
            return carry

        lax.fori_loop(0, (_NCH - 1) // 2, outer, 0)
        c = _NCH - 1
        pltpu.async_copy(eh.at[pl.ds(base + c * _K, _K)], bufs[0],
                         lsem).wait()
        pltpu.sync_copy(bufs[0], acc.at[dstv.at[c]], add=True)
        plsc.subcore_barrier()
        pltpu.sync_copy(acc.at[pl.ds(sid * _RPT, _RPT)],
                        out.at[pl.ds(cid * _N + sid * _RPT, _RPT)])

        @pl.when(sid == 0)
        def _out_tail():
            pltpu.sync_copy(acc.at[pl.ds(_RPT * _NS, _RTAIL)],
                            out.at[pl.ds(cid * _N + _RPT * _NS, _RTAIL)])

    return _sc_scatter


def _vec(x):
    return x.reshape(1, -1)


def _w12(layers, w1):
    out = []
    for i, l in enumerate(layers):
        w = w1 if i == 0 else l['W']
        out += [w, _vec(l['b']), _vec(l['g']), _vec(l['be'])]
    return out


def kernel(X, edge_index, edge_attr, params):
    ea = jnp.pad(edge_attr, ((0, 0), (0, 1)))
    src3d = [edge_index[0, h * _EH:(h + 1) * _EH].reshape(_NW, _NCH, _K)
             for h in range(2)]
    dst3d = [edge_index[1, h * _EH:(h + 1) * _EH].reshape(_NW, _NCH, _K)
             for h in range(2)]
    eah = [ea[h * _EH:(h + 1) * _EH] for h in range(2)]

    ne = params['node_enc']
    ee = params['edge_enc']
    proc = params['proc']
    dec = params['dec']

    e_w1 = [s['edge'][0]['W'] for s in proc]
    n_w1 = [s['node'][0]['W'] for s in proc]
    ee_w1 = jnp.pad(ee[0]['W'], ((0, 1), (0, 0)))

    zeros = jnp.zeros((_N, _D), jnp.float32)

    Xh, P, Q = _rowwise_call(
        _node_enc_body, _N, 1000, [X],
        _w12(ne, ne[0]['W'][0:1, :]) + [e_w1[0][0:_D], e_w1[0][_D:2 * _D]],
        [(_N, _D)] * 3)

    Eh = [None, None]
    for s in range(2):
        G = [_sc_gather_kernel()(src3d[h], dst3d[h], P, Q) for h in range(2)]
        if s == 0:
            ew = _w12(ee, ee_w1) + _w12(proc[0]['edge'],
                                        e_w1[0][2 * _D:3 * _D])
            Eh = [_rowwise_call(_edge_enc_mlp_body, _EH, 4000,
                                [eah[h], G[h]], ew, [(_EH, _D)])
                  for h in range(2)]
        else:
            ew = _w12(proc[s]['edge'], e_w1[s][2 * _D:3 * _D])
            Eh = [_rowwise_call(_edge_mlp_body, _EH, 4000,
                                [G[h], Eh[h]], ew, [(_EH, _D)])
                  for h in range(2)]
        parts = [_sc_scatter_kernel()(Eh[h], dst3d[h], zeros)
                 for h in range(2)]
        aggs = [parts[0][:_N], parts[0][_N:], parts[1][:_N], parts[1][_N:]]
        nw = _w12(proc[s]['node'], n_w1[s][0:_D]) + [n_w1[s][_D:2 * _D]]
        nw = [nw[0], nw[12]] + nw[1:12]
        if s == 0:
            Xh, P, Q = _rowwise_call(
                _node_mlp_body, _N, 1000, [Xh] + aggs,
                nw + [e_w1[1][0:_D], e_w1[1][_D:2 * _D]], [(_N, _D)] * 3)
        else:
            dw = [dec[0]['W'], _vec(dec[0]['b']), _vec(dec[0]['g']),
                  _vec(dec[0]['be']),
                  dec[1]['W'], _vec(dec[1]['b']), _vec(dec[1]['g']),
                  _vec(dec[1]['be']),
                  jnp.pad(dec[2]['W'], ((0, 0), (0, _D - 6))),
                  jnp.pad(_vec(dec[2]['b']), ((0, 0), (0, _D - 6)))]
            Ypad = _rowwise_call(_node_dec_body, _N, 1000, [Xh] + aggs,
                                 nw + dw, [(_N, _D)])
    return Ypad[:, :6]


# speedup vs baseline: 5.3203x; 1.0045x over previous
import functools

import jax
import jax.numpy as jnp
from jax import lax
from jax.experimental import pallas as pl
from jax.experimental.pallas import tpu as pltpu
from jax.experimental.pallas import tpu_sc as plsc

_N = 10000
_E = 320000
_D = 128
_EPS = 1e-5

_NC = 2
_NS = 16
_NW = _NC * _NS
_EH = _E // 2
_EPW = _EH // _NW
_K = 40
_NCH = _EPW // _K
_RPT = 624
_RTAIL = _N - _RPT * _NS
_NB = 5


def _ln(x, g, b):
    n = x.shape[-1]
    o = jnp.full((n, n), 1.0 / n, jnp.float32)
    mu = _dot(x, o)
    xc = x - mu
    var = _dot(xc * xc, o)
    return xc * lax.rsqrt(var + _EPS) * g + b


def _dot(a, w):
    return jnp.dot(a, w, preferred_element_type=jnp.float32)


def _block(h, g, e, relu=True):
    if relu:
        h = jnp.maximum(h, 0.0)
    return _ln(h, g, e)


def _tail2(h, w2, b2, g2, e2, w3, b3, g3, e3):
    h = _block(_dot(h, w2[...]) + b2[...], g2[...], e2[...])
    return _block(_dot(h, w3[...]) + b3[...], g3[...], e3[...], relu=False)


def _edge_enc_body(x_ref, w1, b1, g1, e1, w2, b2, g2, e2, w3, b3, g3, e3,
                   o_ref):
    h = _block(_dot(x_ref[...], w1[...]) + b1[...], g1[...], e1[...])
    o_ref[...] = _tail2(h, w2, b2, g2, e2, w3, b3, g3, e3)


def _node_enc_body(x_ref, w1r, b1, g1, e1, w2, b2, g2, e2, w3, b3, g3, e3,
                   wpa, wpb, xh_ref, p_ref, q_ref):
    x0 = x_ref[...][:, 0:1]
    h = _block(x0 * w1r[...] + b1[...], g1[...], e1[...])
    xh = _tail2(h, w2, b2, g2, e2, w3, b3, g3, e3)
    xh_ref[...] = xh
    p_ref[...] = _dot(xh, wpa[...])
    q_ref[...] = _dot(xh, wpb[...])


def _edge_enc_mlp_body(x_ref, g_ref,
                       ew1, eb1, eg1, ee1, ew2, eb2, eg2, ee2, ew3, eb3, eg3,
                       ee3, w1c, b1, g1, e1, w2, b2, g2, e2, w3, b3, g3, e3,
                       o_ref):
    eh = _block(_dot(x_ref[...], ew1[...]) + eb1[...], eg1[...], ee1[...])
    eh = _tail2(eh, ew2, eb2, eg2, ee2, ew3, eb3, eg3, ee3)
    h = _block(g_ref[...] + _dot(eh, w1c[...]) + b1[...], g1[...], e1[...])
    o_ref[...] = eh + _tail2(h, w2, b2, g2, e2, w3, b3, g3, e3)


def _edge_mlp_body(g_ref, eh_ref, w1c, b1, g1, e1, w2, b2, g2, e2,
                   w3, b3, g3, e3, o_ref):
    eh = eh_ref[...]
    h = _block(g_ref[...] + _dot(eh, w1c[...]) + b1[...],
               g1[...], e1[...])
    o_ref[...] = eh + _tail2(h, w2, b2, g2, e2, w3, b3, g3, e3)


def _node_mlp_body(xh_ref, aa_ref, ab_ref, ba_ref, bb_ref, w1a, w1b, b1, g1,
                   e1, w2, b2, g2, e2, w3, b3, g3, e3, wpa, wpb, xh_ref_o,
                   p_ref, q_ref):
    xh = xh_ref[...]
    agg = ((aa_ref[...] + ab_ref[...]) + (ba_ref[...] + bb_ref[...]))
    h = _block(_dot(xh, w1a[...]) + _dot(agg, w1b[...]) + b1[...],
               g1[...], e1[...])
    xh2 = xh + _tail2(h, w2, b2, g2, e2, w3, b3, g3, e3)
    xh_ref_o[...] = xh2
    p_ref[...] = _dot(xh2, wpa[...])
    q_ref[...] = _dot(xh2, wpb[...])


def _node_dec_body(xh_ref, aa_ref, ab_ref, ba_ref, bb_ref, w1a, w1b, b1, g1,
                   e1, w2, b2, g2, e2, w3, b3, g3, e3, dw1, db1, dg1, de1,
                   dw2, db2, dg2, de2, dw3, db3, y_ref):
    xh = xh_ref[...]
    agg = ((aa_ref[...] + ab_ref[...]) + (ba_ref[...] + bb_ref[...]))
    h = _block(_dot(xh, w1a[...]) + _dot(agg, w1b[...]) + b1[...],
               g1[...], e1[...])
    xh2 = xh + _tail2(h, w2, b2, g2, e2, w3, b3, g3, e3)
    h = _block(_dot(xh2, dw1[...]) + db1[...], dg1[...], de1[...])
    h = _block(_dot(h, dw2[...]) + db2[...], dg2[...], de2[...])
    y_ref[...] = _dot(h, dw3[...]) + db3[...]


def _rowwise_call(body, nrows, rblk, data, weights, out_shapes):
    grid = (nrows // rblk,)
    in_specs = (
        [pl.BlockSpec((rblk, a.shape[1]), lambda i: (i, 0)) for a in data]
        + [pl.BlockSpec(w.shape, functools.partial(lambda nd, i: (0,) * nd,
                                                   w.ndim))
           for w in weights])
    out_specs = [pl.BlockSpec((rblk, s[1]), lambda i: (i, 0))
                 for s in out_shapes]
    out_shape = [jax.ShapeDtypeStruct(s, jnp.float32) for s in out_shapes]
    if len(out_shapes) == 1:
        out_specs, out_shape = out_specs[0], out_shape[0]
    return pl.pallas_call(
        body, grid=grid, in_specs=in_specs, out_specs=out_specs,
        out_shape=out_shape)(*data, *weights)


@functools.cache
def _sc_gather_kernel():
    mesh = plsc.VectorSubcoreMesh(core_axis_name="c", subcore_axis_name="s")

    @functools.partial(
        pl.kernel,
        mesh=mesh,
        out_type=jax.ShapeDtypeStruct((_EH, _D), jnp.float32),
        scratch_types=[pltpu.VMEM((_NCH, _K), jnp.int32),
                       pltpu.VMEM((_NCH, _K), jnp.int32),
                       [pltpu.VMEM((_K, _D), jnp.float32)
                        for _ in range(_NB)],
                       [pltpu.VMEM((_K, _D), jnp.float32)
                        for _ in range(_NB)],
                       pltpu.SemaphoreType.DMA,
                       pltpu.SemaphoreType.DMA],
    )
    def _sc_gather(src3d, dst3d, p, q, g, srcv, dstv, bps, bqs, gsem, ssem):
        wid = lax.axis_index("s") * _NC + lax.axis_index("c")
        base = wid * _EPW
        pltpu.sync_copy(src3d.at[wid], srcv)
        pltpu.sync_copy(dst3d.at[wid], dstv)

        def outer(o, carry):
            c0 = o * _NB
            gd = []
            for b in range(_NB):
                c = c0 + b
                gd.append(pltpu.async_copy(p.at[srcv.at[c]], bps[b], gsem))
                gd.append(pltpu.async_copy(q.at[dstv.at[c]], bqs[b], gsem))
            sd = []
            for b in range(_NB):
                c = c0 + b
                gd[2 * b].wait()
                gd[2 * b + 1].wait()
                bp, bq = bps[b], bqs[b]

                def add_row(r, carry2, bp=bp, bq=bq):
                    for j in range(_D // 16):
                        s = pl.ds(j * 16, 16)
                        bp[r, s] = bp[r, s] + bq[r, s]
                    return carry2

                lax.fori_loop(0, _K, add_row, 0)
                sd.append(pltpu.async_copy(
                    bp, g.at[pl.ds(base + c * _K, _K)], ssem))
            for d in sd:
                d.wait()
            return carry

        lax.fori_loop(0, _NCH // _NB, outer, 0)

    return _sc_gather


@functools.cache
def _sc_scatter_kernel():
    mesh = plsc.VectorSubcoreMesh(core_axis_name="c", subcore_axis_name="s")

    @functools.partial(
        pl.kernel,
        mesh=mesh,
        out_type=jax.ShapeDtypeStruct((2 * _N, _D), jnp.float32),
        scratch_types=[pltpu.VMEM((_NCH, _K), jnp.int32),
                       [pltpu.VMEM((_K, _D), jnp.float32)
                        for _ in range(2)],
                       pltpu.SemaphoreType.DMA,
                       pltpu.SemaphoreType.DMA,
                       pltpu.VMEM_SHARED((_N, _D), jnp.float32)],
    )
    def _sc_scatter(eh, dst3d, zeros, out, dstv, bufs, lsem, asem, acc):
        cid = lax.axis_index("c")
        sid = lax.axis_index("s")
        wid = sid * _NC + cid
        pltpu.sync_copy(zeros.at[pl.ds(sid * _RPT, _RPT)],
                        acc.at[pl.ds(sid * _RPT, _RPT)])

        @pl.when(sid == 0)
        def _init_tail():
            pltpu.sync_copy(zeros.at[pl.ds(_RPT * _NS, _RTAIL)],
                            acc.at[pl.ds(_RPT * _NS, _RTAIL)])

        pltpu.sync_copy(dst3d.at[wid], dstv)
        plsc.subcore_barrier()
        base = wid * _EPW

        def outer(o, carry):
            c0 = o * 2
            ld = []
            for b in range(2):
                c = c0 + b
                ld.append(pltpu.async_copy(
                    eh.at[pl.ds(base + c * _K, _K)], bufs[b], lsem))
            ad = []
            for b in range(2):
                c = c0 + b
                ld[b].wait()
                ad.append(pltpu.async_copy(
                    bufs[b], acc.at[dstv.at[c]], asem, add=True))
            for d in ad:
                d.wait()
            return carry

        lax.fori_loop(0, (_NCH - 1) // 2, outer, 0)
        c = _NCH - 1
        pltpu.async_copy(eh.at[pl.ds(base + c * _K, _K)], bufs[0],
                         lsem).wait()
        pltpu.sync_copy(bufs[0], acc.at[dstv.at[c]], add=True)
        plsc.subcore_barrier()
        pltpu.sync_copy(acc.at[pl.ds(sid * _RPT, _RPT)],
                        out.at[pl.ds(cid * _N + sid * _RPT, _RPT)])

        @pl.when(sid == 0)
        def _out_tail():
            pltpu.sync_copy(acc.at[pl.ds(_RPT * _NS, _RTAIL)],
                            out.at[pl.ds(cid * _N + _RPT * _NS, _RTAIL)])

    return _sc_scatter


def _vec(x):
    return x.reshape(1, -1)


def _w12(layers, w1):
    out = []
    for i, l in enumerate(layers):
        w = w1 if i == 0 else l['W']
        out += [w, _vec(l['b']), _vec(l['g']), _vec(l['be'])]
    return out


def kernel(X, edge_index, edge_attr, params):
    ea = jnp.pad(edge_attr, ((0, 0), (0, 1)))
    src3d = [edge_index[0, h * _EH:(h + 1) * _EH].reshape(_NW, _NCH, _K)
             for h in range(2)]
    dst3d = [edge_index[1, h * _EH:(h + 1) * _EH].reshape(_NW, _NCH, _K)
             for h in range(2)]
    eah = [ea[h * _EH:(h + 1) * _EH] for h in range(2)]

    ne = params['node_enc']
    ee = params['edge_enc']
    proc = params['proc']
    dec = params['dec']

    e_w1 = [s['edge'][0]['W'] for s in proc]
    n_w1 = [s['node'][0]['W'] for s in proc]
    ee_w1 = jnp.pad(ee[0]['W'], ((0, 1), (0, 0)))

    zeros = jnp.zeros((_N, _D), jnp.float32)

    Xh, P, Q = _rowwise_call(
        _node_enc_body, _N, 1000, [X],
        _w12(ne, ne[0]['W'][0:1, :]) + [e_w1[0][0:_D], e_w1[0][_D:2 * _D]],
        [(_N, _D)] * 3)

    Eh = [None, None]
    for s in range(2):
        G = [_sc_gather_kernel()(src3d[h], dst3d[h], P, Q) for h in range(2)]
        if s == 0:
            ew = _w12(ee, ee_w1) + _w12(proc[0]['edge'],
                                        e_w1[0][2 * _D:3 * _D])
            Eh = [_rowwise_call(_edge_enc_mlp_body, _EH, 4000,
                                [eah[h], G[h]], ew, [(_EH, _D)])
                  for h in range(2)]
        else:
            ew = _w12(proc[s]['edge'], e_w1[s][2 * _D:3 * _D])
            Eh = [_rowwise_call(_edge_mlp_body, _EH, 4000,
                                [G[h], Eh[h]], ew, [(_EH, _D)])
                  for h in range(2)]
        parts = [_sc_scatter_kernel()(Eh[h], dst3d[h], zeros)
                 for h in range(2)]
        aggs = [parts[0][:_N], parts[0][_N:], parts[1][:_N], parts[1][_N:]]
        nw = _w12(proc[s]['node'], n_w1[s][0:_D]) + [n_w1[s][_D:2 * _D]]
        nw = [nw[0], nw[12]] + nw[1:12]
        if s == 0:
            Xh, P, Q = _rowwise_call(
                _node_mlp_body, _N, 1000, [Xh] + aggs,
                nw + [e_w1[1][0:_D], e_w1[1][_D:2 * _D]], [(_N, _D)] * 3)
        else:
            dw = [dec[0]['W'], _vec(dec[0]['b']), _vec(dec[0]['g']),
                  _vec(dec[0]['be']),
                  dec[1]['W'], _vec(dec[1]['b']), _vec(dec[1]['g']),
                  _vec(dec[1]['be']),
                  jnp.pad(dec[2]['W'], ((0, 0), (0, _D - 6))),
                  jnp.pad(_vec(dec[2]['b']), ((0, 0), (0, _D - 6)))]
            Ypad = _rowwise_call(_node_dec_body, _N, 1000, [Xh] + aggs,
                                 nw + dw, [(_N, _D)])
    return Ypad[:, :6]


# speedup vs baseline: 5.3471x; 1.0050x over previous
import functools

import jax
import jax.numpy as jnp
from jax import lax
from jax.experimental import pallas as pl
from jax.experimental.pallas import tpu as pltpu
from jax.experimental.pallas import tpu_sc as plsc

_N = 10000
_E = 320000
_D = 128
_EPS = 1e-5

_NC = 2
_NS = 16
_NW = _NC * _NS
_EH = _E // 2
_EPW = _EH // _NW
_K = 40
_NCH = _EPW // _K
_RPT = 624
_RTAIL = _N - _RPT * _NS
_NB = 5


def _ln(x, g, b):
    n = x.shape[-1]
    o = jnp.full((n, n), 1.0 / n, jnp.float32)
    mu = _dot(x, o)
    xc = x - mu
    var = _dot(xc * xc, o)
    return xc * lax.rsqrt(var + _EPS) * g + b


def _dot(a, w):
    return jnp.dot(a, w, preferred_element_type=jnp.float32)


def _block(h, g, e, relu=True):
    if relu:
        h = jnp.maximum(h, 0.0)
    return _ln(h, g, e)


def _tail2(h, w2, b2, g2, e2, w3, b3, g3, e3):
    h = _block(_dot(h, w2[...]) + b2[...], g2[...], e2[...])
    return _block(_dot(h, w3[...]) + b3[...], g3[...], e3[...], relu=False)


def _edge_enc_body(x_ref, w1, b1, g1, e1, w2, b2, g2, e2, w3, b3, g3, e3,
                   o_ref):
    h = _block(_dot(x_ref[...], w1[...]) + b1[...], g1[...], e1[...])
    o_ref[...] = _tail2(h, w2, b2, g2, e2, w3, b3, g3, e3)


def _node_enc_body(x_ref, w1r, b1, g1, e1, w2, b2, g2, e2, w3, b3, g3, e3,
                   wpa, wpb, xh_ref, p_ref, q_ref):
    x0 = x_ref[...][:, 0:1]
    h = _block(x0 * w1r[...] + b1[...], g1[...], e1[...])
    xh = _tail2(h, w2, b2, g2, e2, w3, b3, g3, e3)
    xh_ref[...] = xh
    p_ref[...] = _dot(xh, wpa[...])
    q_ref[...] = _dot(xh, wpb[...])


def _edge_enc_mlp_body(x_ref, g_ref,
                       ew1, eb1, eg1, ee1, ew2, eb2, eg2, ee2, ew3, eb3, eg3,
                       ee3, w1c, b1, g1, e1, w2, b2, g2, e2, w3, b3, g3, e3,
                       o_ref):
    eh = _block(_dot(x_ref[...], ew1[...]) + eb1[...], eg1[...], ee1[...])
    eh = _tail2(eh, ew2, eb2, eg2, ee2, ew3, eb3, eg3, ee3)
    h = _block(g_ref[...] + _dot(eh, w1c[...]) + b1[...], g1[...], e1[...])
    o_ref[...] = eh + _tail2(h, w2, b2, g2, e2, w3, b3, g3, e3)


def _edge_mlp_body(g_ref, eh_ref, w1c, b1, g1, e1, w2, b2, g2, e2,
                   w3, b3, g3, e3, o_ref):
    eh = eh_ref[...]
    h = _block(g_ref[...] + _dot(eh, w1c[...]) + b1[...],
               g1[...], e1[...])
    o_ref[...] = eh + _tail2(h, w2, b2, g2, e2, w3, b3, g3, e3)


def _node_mlp_body(xh_ref, aa_ref, ab_ref, ba_ref, bb_ref, w1a, w1b, b1, g1,
                   e1, w2, b2, g2, e2, w3, b3, g3, e3, wpa, wpb, xh_ref_o,
                   p_ref, q_ref):
    xh = xh_ref[...]
    agg = ((aa_ref[...] + ab_ref[...]) + (ba_ref[...] + bb_ref[...]))
    h = _block(_dot(xh, w1a[...]) + _dot(agg, w1b[...]) + b1[...],
               g1[...], e1[...])
    xh2 = xh + _tail2(h, w2, b2, g2, e2, w3, b3, g3, e3)
    xh_ref_o[...] = xh2
    p_ref[...] = _dot(xh2, wpa[...])
    q_ref[...] = _dot(xh2, wpb[...])


def _node_dec_body(xh_ref, aa_ref, ab_ref, ba_ref, bb_ref, w1a, w1b, b1, g1,
                   e1, w2, b2, g2, e2, w3, b3, g3, e3, dw1, db1, dg1, de1,
                   dw2, db2, dg2, de2, dw3, db3, y_ref):
    xh = xh_ref[...]
    agg = ((aa_ref[...] + ab_ref[...]) + (ba_ref[...] + bb_ref[...]))
    h = _block(_dot(xh, w1a[...]) + _dot(agg, w1b[...]) + b1[...],
               g1[...], e1[...])
    xh2 = xh + _tail2(h, w2, b2, g2, e2, w3, b3, g3, e3)
    h = _block(_dot(xh2, dw1[...]) + db1[...], dg1[...], de1[...])
    h = _block(_dot(h, dw2[...]) + db2[...], dg2[...], de2[...])
    y_ref[...] = _dot(h, dw3[...]) + db3[...]


def _rowwise_call(body, nrows, rblk, data, weights, out_shapes):
    grid = (nrows // rblk,)
    in_specs = (
        [pl.BlockSpec((rblk, a.shape[1]), lambda i: (i, 0)) for a in data]
        + [pl.BlockSpec(w.shape, functools.partial(lambda nd, i: (0,) * nd,
                                                   w.ndim))
           for w in weights])
    out_specs = [pl.BlockSpec((rblk, s[1]), lambda i: (i, 0))
                 for s in out_shapes]
    out_shape = [jax.ShapeDtypeStruct(s, jnp.float32) for s in out_shapes]
    if len(out_shapes) == 1:
        out_specs, out_shape = out_specs[0], out_shape[0]
    return pl.pallas_call(
        body, grid=grid, in_specs=in_specs, out_specs=out_specs,
        out_shape=out_shape)(*data, *weights)


@functools.cache
def _sc_gather_kernel():
    mesh = plsc.VectorSubcoreMesh(core_axis_name="c", subcore_axis_name="s")

    @functools.partial(
        pl.kernel,
        mesh=mesh,
        out_type=jax.ShapeDtypeStruct((_EH, _D), jnp.float32),
        scratch_types=[pltpu.VMEM((_NCH, _K), jnp.int32),
                       pltpu.VMEM((_NCH, _K), jnp.int32),
                       [pltpu.VMEM((_K, _D), jnp.float32)
                        for _ in range(_NB)],
                       [pltpu.VMEM((_K, _D), jnp.float32)
                        for _ in range(_NB)],
                       pltpu.SemaphoreType.DMA,
                       pltpu.SemaphoreType.DMA],
    )
    def _sc_gather(src3d, dst3d, p, q, g, srcv, dstv, bps, bqs, gsem, ssem):
        wid = lax.axis_index("s") * _NC + lax.axis_index("c")
        base = wid * _EPW
        pltpu.sync_copy(src3d.at[wid], srcv)
        pltpu.sync_copy(dst3d.at[wid], dstv)

        def outer(o, carry):
            c0 = o * _NB
            gd = []
            for b in range(_NB):
                c = c0 + b
                gd.append(pltpu.async_copy(p.at[srcv.at[c]], bps[b], gsem))
                gd.append(pltpu.async_copy(q.at[dstv.at[c]], bqs[b], gsem))
            sd = []
            for b in range(_NB):
                c = c0 + b
                gd[2 * b].wait()
                gd[2 * b + 1].wait()
                bp, bq = bps[b], bqs[b]

                def add_row(r, carry2, bp=bp, bq=bq):
                    for j in range(_D // 16):
                        s = pl.ds(j * 16, 16)
                        bp[r, s] = bp[r, s] + bq[r, s]
                    return carry2

                lax.fori_loop(0, _K, add_row, 0)
                sd.append(pltpu.async_copy(
                    bp, g.at[pl.ds(base + c * _K, _K)], ssem))
            for d in sd:
                d.wait()
            return carry

        lax.fori_loop(0, _NCH // _NB, outer, 0)

    return _sc_gather


@functools.cache
def _sc_scatter_kernel():
    mesh = plsc.VectorSubcoreMesh(core_axis_name="c", subcore_axis_name="s")

    @functools.partial(
        pl.kernel,
        mesh=mesh,
        out_type=jax.ShapeDtypeStruct((2 * _N, _D), jnp.float32),
        scratch_types=[pltpu.VMEM((_NCH, _K), jnp.int32),
                       [pltpu.VMEM((_K, _D), jnp.float32)
                        for _ in range(2)],
                       pltpu.SemaphoreType.DMA,
                       pltpu.SemaphoreType.DMA,
                       pltpu.VMEM_SHARED((_N, _D), jnp.float32)],
    )
    def _sc_scatter(eh, dst3d, zeros, out, dstv, bufs, lsem, asem, acc):
        cid = lax.axis_index("c")
        sid = lax.axis_index("s")
        wid = sid * _NC + cid
        pltpu.sync_copy(zeros.at[pl.ds(sid * _RPT, _RPT)],
                        acc.at[pl.ds(sid * _RPT, _RPT)])

        @pl.when(sid == 0)
        def _init_tail():
            pltpu.sync_copy(zeros.at[pl.ds(_RPT * _NS, _RTAIL)],
                            acc.at[pl.ds(_RPT * _NS, _RTAIL)])

        pltpu.sync_copy(dst3d.at[wid], dstv)
        plsc.subcore_barrier()
        base = wid * _EPW

        def outer(o, carry):
            c0 = o * 2
            ld = []
            for b in range(2):
                c = c0 + b
                ld.append(pltpu.async_copy(
                    eh.at[pl.ds(base + c * _K, _K)], bufs[b], lsem))
            ad = []
            for b in range(2):
                c = c0 + b
                ld[b].wait()
                ad.append(pltpu.async_copy(
                    bufs[b], acc.at[dstv.at[c]], asem, add=True))
            for d in ad:
                d.wait()
            return carry

        lax.fori_loop(0, (_NCH - 1) // 2, outer, 0)
        c = _NCH - 1
        pltpu.async_copy(eh.at[pl.ds(base + c * _K, _K)], bufs[0],
                         lsem).wait()
        pltpu.sync_copy(bufs[0], acc.at[dstv.at[c]], add=True)
        plsc.subcore_barrier()
        pltpu.sync_copy(acc.at[pl.ds(sid * _RPT, _RPT)],
                        out.at[pl.ds(cid * _N + sid * _RPT, _RPT)])

        @pl.when(sid == 0)
        def _out_tail():
            pltpu.sync_copy(acc.at[pl.ds(_RPT * _NS, _RTAIL)],
                            out.at[pl.ds(cid * _N + _RPT * _NS, _RTAIL)])

    return _sc_scatter


def _vec(x):
    return x.reshape(1, -1)


def _w12(layers, w1):
    out = []
    for i, l in enumerate(layers):
        w = w1 if i == 0 else l['W']
        out += [w, _vec(l['b']), _vec(l['g']), _vec(l['be'])]
    return out


def kernel(X, edge_index, edge_attr, params):
    ea = jnp.pad(edge_attr, ((0, 0), (0, 1)))
    src3d = [edge_index[0, h * _EH:(h + 1) * _EH].reshape(_NW, _NCH, _K)
             for h in range(2)]
    dst3d = [edge_index[1, h * _EH:(h + 1) * _EH].reshape(_NW, _NCH, _K)
             for h in range(2)]
    eah = [ea[h * _EH:(h + 1) * _EH] for h in range(2)]

    ne = params['node_enc']
    ee = params['edge_enc']
    proc = params['proc']
    dec = params['dec']

    e_w1 = [s['edge'][0]['W'] for s in proc]
    n_w1 = [s['node'][0]['W'] for s in proc]
    ee_w1 = jnp.pad(ee[0]['W'], ((0, 1), (0, 0)))

    zeros = jnp.zeros((_N, _D), jnp.float32)

    Xh, P, Q = _rowwise_call(
        _node_enc_body, _N, 1000, [X],
        _w12(ne, ne[0]['W'][0:1, :]) + [e_w1[0][0:_D], e_w1[0][_D:2 * _D]],
        [(_N, _D)] * 3)

    Eh = [None, None]
    for s in range(2):
        G = [_sc_gather_kernel()(src3d[h], dst3d[h], P, Q) for h in range(2)]
        if s == 0:
            ew = _w12(ee, ee_w1) + _w12(proc[0]['edge'],
                                        e_w1[0][2 * _D:3 * _D])
            Eh = [_rowwise_call(_edge_enc_mlp_body, _EH, 8000,
                                [eah[h], G[h]], ew, [(_EH, _D)])
                  for h in range(2)]
        else:
            ew = _w12(proc[s]['edge'], e_w1[s][2 * _D:3 * _D])
            Eh = [_rowwise_call(_edge_mlp_body, _EH, 8000,
                                [G[h], Eh[h]], ew, [(_EH, _D)])
                  for h in range(2)]
        parts = [_sc_scatter_kernel()(Eh[h], dst3d[h], zeros)
                 for h in range(2)]
        aggs = [parts[0][:_N], parts[0][_N:], parts[1][:_N], parts[1][_N:]]
        nw = _w12(proc[s]['node'], n_w1[s][0:_D]) + [n_w1[s][_D:2 * _D]]
        nw = [nw[0], nw[12]] + nw[1:12]
        if s == 0:
            Xh, P, Q = _rowwise_call(
                _node_mlp_body, _N, 1000, [Xh] + aggs,
                nw + [e_w1[1][0:_D], e_w1[1][_D:2 * _D]], [(_N, _D)] * 3)
        else:
            dw = [dec[0]['W'], _vec(dec[0]['b']), _vec(dec[0]['g']),
                  _vec(dec[0]['be']),
                  dec[1]['W'], _vec(dec[1]['b']), _vec(dec[1]['g']),
                  _vec(dec[1]['be']),
                  jnp.pad(dec[2]['W'], ((0, 0), (0, _D - 6))),
                  jnp.pad(_vec(dec[2]['b']), ((0, 0), (0, _D - 6)))]
            Ypad = _rowwise_call(_node_dec_body, _N, 1000, [Xh] + aggs,
                                 nw + dw, [(_N, _D)])
    return Ypad[:, :6]
